# speedup vs baseline: 1.6056x; 1.6056x over previous
import functools

import jax
import jax.numpy as jnp
from jax import lax
from jax.experimental import pallas as pl
from jax.experimental.pallas import tpu as pltpu
from jax.experimental.pallas import tpu_sc as plsc

N = 100000
D = 128
VOCAB_PAD = 84
CHUNK = 80
NCHUNK = N // CHUNK
NC = 2
NS = 16
NW = NC * NS

_mesh = plsc.VectorSubcoreMesh(core_axis_name="c", subcore_axis_name="s")


@functools.partial(
    pl.kernel,
    mesh=_mesh,
    out_type=jax.ShapeDtypeStruct((N, D), jnp.float32),
    scratch_types=[
        pltpu.VMEM((CHUNK,), jnp.int32),
        pltpu.VMEM((CHUNK, D), jnp.float32),
        pltpu.SemaphoreType.DMA,
    ],
)
def _emb(table_hbm, idx_hbm, out_hbm, idx_v, rows_v, sem):
    wid = lax.axis_index("s") * NC + lax.axis_index("c")
    iters = (NCHUNK + NW - 1) // NW

    def body(i, carry):
        c = wid + i * NW

        @pl.when(c < NCHUNK)
        def _():
            base = c * CHUNK
            pltpu.sync_copy(idx_hbm.at[pl.ds(base, CHUNK)], idx_v)
            pltpu.async_copy(table_hbm.at[idx_v], rows_v, sem).wait()
            pltpu.sync_copy(rows_v, out_hbm.at[pl.ds(base, CHUNK)])

        return carry

    lax.fori_loop(0, iters, body, 0)


def kernel(Z, W):
    W_pad = jnp.concatenate([jnp.zeros((1, D), W.dtype), W], axis=0)
    return _emb(W_pad, Z.astype(jnp.int32))


# speedup vs baseline: 1.6627x; 1.0356x over previous
import functools

import jax
import jax.numpy as jnp
from jax import lax
from jax.experimental import pallas as pl
from jax.experimental.pallas import tpu as pltpu
from jax.experimental.pallas import tpu_sc as plsc

N = 100000
D = 128
VOCAB_PAD = 84
CHUNK = 400
NCHUNK = N // CHUNK
NC = 2
NS = 16
NW = NC * NS

_mesh = plsc.VectorSubcoreMesh(core_axis_name="c", subcore_axis_name="s")


@functools.partial(
    pl.kernel,
    mesh=_mesh,
    out_type=jax.ShapeDtypeStruct((N, D), jnp.float32),
    scratch_types=[
        pltpu.VMEM((CHUNK,), jnp.int32),
        pltpu.VMEM((CHUNK, D), jnp.float32),
        pltpu.SemaphoreType.DMA,
    ],
)
def _emb(table_hbm, idx_hbm, out_hbm, idx_v, rows_v, sem):
    wid = lax.axis_index("s") * NC + lax.axis_index("c")
    iters = (NCHUNK + NW - 1) // NW

    def body(i, carry):
        c = wid + i * NW

        @pl.when(c < NCHUNK)
        def _():
            base = c * CHUNK
            pltpu.sync_copy(idx_hbm.at[pl.ds(base, CHUNK)], idx_v)
            pltpu.async_copy(table_hbm.at[idx_v], rows_v, sem).wait()
            pltpu.sync_copy(rows_v, out_hbm.at[pl.ds(base, CHUNK)])

        return carry

    lax.fori_loop(0, iters, body, 0)


def kernel(Z, W):
    W_pad = jnp.concatenate([jnp.zeros((1, D), W.dtype), W], axis=0)
    return _emb(W_pad, Z.astype(jnp.int32))


# speedup vs baseline: 1.6675x; 1.0029x over previous
import functools

import jax
import jax.numpy as jnp
from jax import lax
from jax.experimental import pallas as pl
from jax.experimental.pallas import tpu as pltpu
from jax.experimental.pallas import tpu_sc as plsc

N = 100000
D = 128
VOCAB_PAD = 84
CHUNK = 400
NCHUNK = N // CHUNK
NC = 2
NS = 16
NW = NC * NS
ITERS = (NCHUNK + NW - 1) // NW

_mesh = plsc.VectorSubcoreMesh(core_axis_name="c", subcore_axis_name="s")


@functools.partial(
    pl.kernel,
    mesh=_mesh,
    out_type=jax.ShapeDtypeStruct((N, D), jnp.float32),
    scratch_types=[
        pltpu.VMEM((CHUNK,), jnp.int32),
        pltpu.VMEM((CHUNK,), jnp.int32),
        pltpu.VMEM((CHUNK, D), jnp.float32),
        pltpu.VMEM((CHUNK, D), jnp.float32),
        pltpu.SemaphoreType.DMA,
        pltpu.SemaphoreType.DMA,
        pltpu.SemaphoreType.DMA,
        pltpu.SemaphoreType.DMA,
        pltpu.SemaphoreType.DMA,
        pltpu.SemaphoreType.DMA,
    ],
)
def _emb(table_hbm, idx_hbm, out_hbm,
         ibuf0, ibuf1, rbuf0, rbuf1, si0, si1, sg0, sg1, sw0, sw1):
    ibuf = (ibuf0, ibuf1)
    rbuf = (rbuf0, rbuf1)
    si = (si0, si1)
    sg = (sg0, sg1)
    sw = (sw0, sw1)
    wid = lax.axis_index("s") * NC + lax.axis_index("c")

    for i in range(2):
        c = wid + i * NW

        @pl.when(c < NCHUNK)
        def _(i=i, c=c):
            pltpu.async_copy(idx_hbm.at[pl.ds(c * CHUNK, CHUNK)], ibuf[i], si[i])

    for i in range(ITERS):
        b = i & 1
        c = wid + i * NW

        @pl.when(c < NCHUNK)
        def _(b=b, c=c, i=i):
            base = c * CHUNK
            pltpu.make_async_copy(
                idx_hbm.at[pl.ds(base, CHUNK)], ibuf[b], si[b]).wait()
            if i >= 2:
                pltpu.make_async_copy(
                    rbuf[b], out_hbm.at[pl.ds(base, CHUNK)], sw[b]).wait()
            pltpu.async_copy(table_hbm.at[ibuf[b]], rbuf[b], sg[b]).wait()
            pltpu.async_copy(rbuf[b], out_hbm.at[pl.ds(base, CHUNK)], sw[b])

        c2 = wid + (i + 2) * NW

        @pl.when(c2 < NCHUNK)
        def _(b=b, c2=c2):
            pltpu.async_copy(idx_hbm.at[pl.ds(c2 * CHUNK, CHUNK)], ibuf[b], si[b])

    for i in range(max(ITERS - 2, 0), ITERS):
        b = i & 1
        c = wid + i * NW

        @pl.when(c < NCHUNK)
        def _(b=b, c=c):
            pltpu.make_async_copy(
                rbuf[b], out_hbm.at[pl.ds(c * CHUNK, CHUNK)], sw[b]).wait()


def kernel(Z, W):
    W_pad = jnp.concatenate([jnp.zeros((1, D), W.dtype), W], axis=0)
    return _emb(W_pad, Z.astype(jnp.int32))


# speedup vs baseline: 5.5474x; 3.3268x over previous
import functools

import jax
import jax.numpy as jnp
from jax import lax
from jax.experimental import pallas as pl
from jax.experimental.pallas import tpu as pltpu
from jax.experimental.pallas import tpu_sc as plsc

N = 100000
D = 128
VOCAB_PAD = 84
CHUNK = 400
NCHUNK = N // CHUNK
NC = 2
NS = 16
NW = NC * NS
ITERS = (NCHUNK + NW - 1) // NW

_mesh = plsc.VectorSubcoreMesh(core_axis_name="c", subcore_axis_name="s")


@functools.partial(
    pl.kernel,
    mesh=_mesh,
    out_type=jax.ShapeDtypeStruct((N, D), jnp.float32),
    scratch_types=[
        pltpu.VMEM((CHUNK,), jnp.int32),
        pltpu.VMEM((CHUNK,), jnp.int32),
        pltpu.VMEM((CHUNK, D), jnp.float32),
        pltpu.VMEM((CHUNK, D), jnp.float32),
        pltpu.VMEM_SHARED((VOCAB_PAD, D), jnp.float32),
        pltpu.SemaphoreType.DMA,
        pltpu.SemaphoreType.DMA,
        pltpu.SemaphoreType.DMA,
        pltpu.SemaphoreType.DMA,
        pltpu.SemaphoreType.DMA,
        pltpu.SemaphoreType.DMA,
    ],
)
def _emb(table_hbm, idx_hbm, out_hbm,
         ibuf0, ibuf1, rbuf0, rbuf1, table_sp, si0, si1, sg0, sg1, sw0, sw1):
    ibuf = (ibuf0, ibuf1)
    rbuf = (rbuf0, rbuf1)
    si = (si0, si1)
    sg = (sg0, sg1)
    sw = (sw0, sw1)
    wid = lax.axis_index("s") * NC + lax.axis_index("c")

    @pl.when(lax.axis_index("s") == 0)
    def _():
        pltpu.sync_copy(table_hbm, table_sp)

    plsc.subcore_barrier()

    for i in range(2):
        c = wid + i * NW

        @pl.when(c < NCHUNK)
        def _(i=i, c=c):
            pltpu.async_copy(idx_hbm.at[pl.ds(c * CHUNK, CHUNK)], ibuf[i], si[i])

    for i in range(ITERS):
        b = i & 1
        c = wid + i * NW

        @pl.when(c < NCHUNK)
        def _(b=b, c=c, i=i):
            base = c * CHUNK
            pltpu.make_async_copy(
                idx_hbm.at[pl.ds(base, CHUNK)], ibuf[b], si[b]).wait()
            if i >= 2:
                pltpu.make_async_copy(
                    rbuf[b], out_hbm.at[pl.ds(base, CHUNK)], sw[b]).wait()
            pltpu.async_copy(table_sp.at[ibuf[b]], rbuf[b], sg[b]).wait()
            pltpu.async_copy(rbuf[b], out_hbm.at[pl.ds(base, CHUNK)], sw[b])

        c2 = wid + (i + 2) * NW

        @pl.when(c2 < NCHUNK)
        def _(b=b, c2=c2):
            pltpu.async_copy(idx_hbm.at[pl.ds(c2 * CHUNK, CHUNK)], ibuf[b], si[b])

    for i in range(max(ITERS - 2, 0), ITERS):
        b = i & 1
        c = wid + i * NW

        @pl.when(c < NCHUNK)
        def _(b=b, c=c):
            pltpu.make_async_copy(
                rbuf[b], out_hbm.at[pl.ds(c * CHUNK, CHUNK)], sw[b]).wait()


def kernel(Z, W):
    W_pad = jnp.concatenate([jnp.zeros((1, D), W.dtype), W], axis=0)
    return _emb(W_pad, Z.astype(jnp.int32))


# speedup vs baseline: 5.6154x; 1.0123x over previous
import functools

import jax
import jax.numpy as jnp
from jax import lax
from jax.experimental import pallas as pl
from jax.experimental.pallas import tpu as pltpu
from jax.experimental.pallas import tpu_sc as plsc

N = 100000
D = 128
VOCAB_PAD = 84
CHUNK = 400
NCHUNK = N // CHUNK
NC = 2
NS = 16
NW = NC * NS
ITERS = (NCHUNK + NW - 1) // NW

_mesh = plsc.VectorSubcoreMesh(core_axis_name="c", subcore_axis_name="s")


@functools.partial(
    pl.kernel,
    mesh=_mesh,
    out_type=jax.ShapeDtypeStruct((N, D), jnp.float32),
    scratch_types=[
        pltpu.VMEM((CHUNK,), jnp.int32),
        pltpu.VMEM((CHUNK,), jnp.int32),
        pltpu.VMEM((CHUNK, D), jnp.float32),
        pltpu.VMEM((CHUNK, D), jnp.float32),
        pltpu.VMEM_SHARED((VOCAB_PAD, D), jnp.float32),
        pltpu.SemaphoreType.DMA,
        pltpu.SemaphoreType.DMA,
        pltpu.SemaphoreType.DMA,
        pltpu.SemaphoreType.DMA,
        pltpu.SemaphoreType.DMA,
        pltpu.SemaphoreType.DMA,
    ],
)
def _emb(table_hbm, idx_hbm, out_hbm,
         ibuf0, ibuf1, rbuf0, rbuf1, table_sp, si0, si1, sg0, sg1, sw0, sw1):
    ibuf = (ibuf0, ibuf1)
    rbuf = (rbuf0, rbuf1)
    si = (si0, si1)
    sg = (sg0, sg1)
    sw = (sw0, sw1)
    wid = lax.axis_index("s") * NC + lax.axis_index("c")

    for i in range(2):
        c = wid + i * NW

        @pl.when(c < NCHUNK)
        def _(i=i, c=c):
            pltpu.async_copy(idx_hbm.at[pl.ds(c * CHUNK, CHUNK)], ibuf[i], si[i])

    @pl.when(lax.axis_index("s") == 0)
    def _():
        pltpu.sync_copy(table_hbm, table_sp.at[pl.ds(1, VOCAB_PAD - 1)])

    plsc.subcore_barrier()

    for i in range(ITERS):
        b = i & 1
        c = wid + i * NW

        @pl.when(c < NCHUNK)
        def _(b=b, c=c, i=i):
            base = c * CHUNK
            pltpu.make_async_copy(
                idx_hbm.at[pl.ds(base, CHUNK)], ibuf[b], si[b]).wait()
            if i >= 2:
                pltpu.make_async_copy(
                    rbuf[b], out_hbm.at[pl.ds(base, CHUNK)], sw[b]).wait()
            pltpu.async_copy(table_sp.at[ibuf[b]], rbuf[b], sg[b]).wait()
            pltpu.async_copy(rbuf[b], out_hbm.at[pl.ds(base, CHUNK)], sw[b])

        c2 = wid + (i + 2) * NW

        @pl.when(c2 < NCHUNK)
        def _(b=b, c2=c2):
            pltpu.async_copy(idx_hbm.at[pl.ds(c2 * CHUNK, CHUNK)], ibuf[b], si[b])

    for i in range(max(ITERS - 2, 0), ITERS):
        b = i & 1
        c = wid + i * NW

        @pl.when(c < NCHUNK)
        def _(b=b, c=c):
            pltpu.make_async_copy(
                rbuf[b], out_hbm.at[pl.ds(c * CHUNK, CHUNK)], sw[b]).wait()


def kernel(Z, W):
    return _emb(W, Z.astype(jnp.int32))


# speedup vs baseline: 5.7059x; 1.0161x over previous
import functools

import jax
import jax.numpy as jnp
from jax import lax
from jax.experimental import pallas as pl
from jax.experimental.pallas import tpu as pltpu
from jax.experimental.pallas import tpu_sc as plsc

N = 100000
D = 128
VOCAB_PAD = 84
CHUNK = 400
NCHUNK = N // CHUNK
NC = 2
NS = 16
NW = NC * NS
ITERS = (NCHUNK + NW - 1) // NW

_mesh = plsc.VectorSubcoreMesh(core_axis_name="c", subcore_axis_name="s")


@functools.partial(
    pl.kernel,
    mesh=_mesh,
    out_type=jax.ShapeDtypeStruct((N, D), jnp.float32),
    scratch_types=[
        pltpu.VMEM((CHUNK,), jnp.int32),
        pltpu.VMEM((CHUNK,), jnp.int32),
        pltpu.VMEM((CHUNK, D), jnp.float32),
        pltpu.VMEM((CHUNK, D), jnp.float32),
        pltpu.VMEM_SHARED((VOCAB_PAD, D), jnp.float32),
        pltpu.SemaphoreType.DMA,
        pltpu.SemaphoreType.DMA,
        pltpu.SemaphoreType.DMA,
        pltpu.SemaphoreType.DMA,
        pltpu.SemaphoreType.DMA,
        pltpu.SemaphoreType.DMA,
    ],
)
def _emb(table_hbm, idx_hbm, out_hbm,
         ibuf0, ibuf1, rbuf0, rbuf1, table_sp, si0, si1, sg0, sg1, sw0, sw1):
    ibuf = (ibuf0, ibuf1)
    rbuf = (rbuf0, rbuf1)
    si = (si0, si1)
    sg = (sg0, sg1)
    sw = (sw0, sw1)
    wid = lax.axis_index("s") * NC + lax.axis_index("c")

    for i in range(2):
        c = wid + i * NW

        @pl.when(c < NCHUNK)
        def _(i=i, c=c):
            pltpu.async_copy(idx_hbm.at[pl.ds(c * CHUNK, CHUNK)], ibuf[i], si[i])

    @pl.when(lax.axis_index("s") == 0)
    def _():
        pltpu.sync_copy(table_hbm, table_sp.at[pl.ds(1, VOCAB_PAD - 1)])

    plsc.subcore_barrier()

    def step(i, carry):
        c = wid + i * NW

        def run(b):
            @pl.when(c < NCHUNK)
            def _():
                base = c * CHUNK
                pltpu.make_async_copy(
                    idx_hbm.at[pl.ds(base, CHUNK)], ibuf[b], si[b]).wait()

                @pl.when(i >= 2)
                def _():
                    pltpu.make_async_copy(
                        rbuf[b], out_hbm.at[pl.ds(base, CHUNK)], sw[b]).wait()

                pltpu.async_copy(table_sp.at[ibuf[b]], rbuf[b], sg[b]).wait()
                pltpu.async_copy(rbuf[b], out_hbm.at[pl.ds(base, CHUNK)], sw[b])

            c2 = wid + (i + 2) * NW

            @pl.when(c2 < NCHUNK)
            def _():
                pltpu.async_copy(
                    idx_hbm.at[pl.ds(c2 * CHUNK, CHUNK)], ibuf[b], si[b])

        @pl.when(i % 2 == 0)
        def _():
            run(0)

        @pl.when(i % 2 == 1)
        def _():
            run(1)

        return carry

    lax.fori_loop(0, ITERS, step, 0)

    for i in range(max(ITERS - 2, 0), ITERS):
        b = i & 1
        c = wid + i * NW

        @pl.when(c < NCHUNK)
        def _(b=b, c=c):
            pltpu.make_async_copy(
                rbuf[b], out_hbm.at[pl.ds(c * CHUNK, CHUNK)], sw[b]).wait()


def kernel(Z, W):
    return _emb(W, Z.astype(jnp.int32))
